# baseline (speedup 1.0000x reference)
import functools

import jax
import jax.numpy as jnp
from jax.experimental import pallas as pl

N = 50000
N_OUT = 100000


def _sparse_conv(feats, W, in_idx, out_idx, n_out):
    g = jnp.take(feats, in_idx.reshape(-1), axis=0).reshape(in_idx.shape + (feats.shape[1],))
    m = jnp.einsum('kpc,kcd->kpd', g, W)
    out = jnp.zeros((n_out, W.shape[-1]), dtype=feats.dtype)
    return out.at[out_idx.reshape(-1)].add(m.reshape(-1, W.shape[-1]))


def _stats_kernel(x_ref, s_ref, q_ref, *, leaky):
    x = x_ref[...]
    if leaky:
        x = jnp.where(x >= 0, x, 0.01 * x)
    s_ref[0, ...] = jnp.sum(x, axis=0, keepdims=True)
    q_ref[0, ...] = jnp.sum(x * x, axis=0, keepdims=True)


def _apply_kernel(x_ref, mu_ref, iv_ref, g_ref, b_ref, o_ref, *, leaky):
    x = x_ref[...]
    if leaky:
        x = jnp.where(x >= 0, x, 0.01 * x)
    o_ref[...] = (x - mu_ref[...]) * iv_ref[...] * g_ref[...] + b_ref[...]


def _bn(x, g, b, leaky=False):
    n, c = x.shape
    bs = 5000
    nb = n // bs
    s, q = pl.pallas_call(
        functools.partial(_stats_kernel, leaky=leaky),
        grid=(nb,),
        in_specs=[pl.BlockSpec((bs, c), lambda i: (i, 0))],
        out_specs=[pl.BlockSpec((1, 1, c), lambda i: (i, 0, 0))] * 2,
        out_shape=[jax.ShapeDtypeStruct((nb, 1, c), x.dtype)] * 2,
    )(x)
    mu = jnp.sum(s, 0) / n
    var = jnp.sum(q, 0) / n - mu * mu
    iv = jax.lax.rsqrt(var + 1e-5)
    return pl.pallas_call(
        functools.partial(_apply_kernel, leaky=leaky),
        grid=(nb,),
        in_specs=[pl.BlockSpec((bs, c), lambda i: (i, 0))] + [pl.BlockSpec((1, c), lambda i: (0, 0))] * 4,
        out_specs=pl.BlockSpec((bs, c), lambda i: (i, 0)),
        out_shape=jax.ShapeDtypeStruct((n, c), x.dtype),
    )(x, mu, iv, g.reshape(1, -1), b.reshape(1, -1))


def kernel(x_features, skip_features, W_trans, bn_t_g, bn_t_b, W1, bn1_g, bn1_b, W2, bn2_g, bn2_b, W_up, rb_trans_in, rb_trans_out, rb1_in, rb1_out, rb2_in, rb2_out, rb_up_in, rb_up_out):
    x = x_features + skip_features
    upA = _sparse_conv(x, W_trans, rb_trans_in, rb_trans_out, N)
    upA = _bn(upA, bn_t_g, bn_t_b, leaky=True)
    upE1 = _bn(_sparse_conv(upA, W1, rb1_in, rb1_out, N), bn1_g, bn1_b)
    upE2 = _bn(_sparse_conv(upA, W2, rb2_in, rb2_out, N), bn2_g, bn2_b)
    upE = upE1 + upE2
    return _sparse_conv(upE, W_up, rb_up_in, rb_up_out, N_OUT)


# speedup vs baseline: 1.7609x; 1.7609x over previous
import functools

import jax
import jax.numpy as jnp
from jax import lax
from jax.experimental import pallas as pl
from jax.experimental.pallas import tpu as pltpu
from jax.experimental.pallas import tpu_sc as plsc

N = 50000
N_OUT = 100000
P = 12500
PP = 12800
C_IN = 128
C_OUT = 64
NC = 2
NS = 16
NW = NC * NS
CH_G = 120
EPS = 1e-5
SBLK = 5000
NB = N // SBLK

_mesh = plsc.VectorSubcoreMesh(core_axis_name="c", subcore_axis_name="s")


def _gather(table, idx):
    B = idx.shape[0]
    D = table.shape[-1]
    per_w = B // NW
    nch = per_w // CH_G

    @functools.partial(
        pl.kernel,
        out_type=jax.ShapeDtypeStruct((B, D), jnp.float32),
        mesh=_mesh,
        scratch_types=[
            pltpu.VMEM((CH_G,), jnp.int32),
            pltpu.VMEM((CH_G, D), jnp.float32),
            pltpu.SemaphoreType.DMA,
        ],
    )
    def k(table_h, idx_h, out_h, idx_v, rows_v, sem):
        wid = lax.axis_index("s") * NC + lax.axis_index("c")
        base = wid * per_w

        def body(i, carry):
            off = pl.multiple_of(base + i * CH_G, 8)
            pltpu.sync_copy(idx_h.at[pl.ds(off, CH_G)], idx_v)
            pltpu.async_copy(table_h.at[idx_v], rows_v, sem).wait()
            pltpu.sync_copy(rows_v, out_h.at[pl.ds(off, CH_G)])
            return carry

        lax.fori_loop(0, nch, body, 0)

    return k(table, idx)


def _add_kernel(a_ref, b_ref, o_ref):
    o_ref[...] = a_ref[...] + b_ref[...]


def _stats_repack_kernel(x_ref, s_ref, q_ref, o_ref):
    x = x_ref[...]
    lx = jnp.where(x >= 0, x, 0.01 * x)
    s_ref[0, ...] = jnp.sum(lx, axis=0, keepdims=True)
    q_ref[0, ...] = jnp.sum(lx * lx, axis=0, keepdims=True)
    o_ref[...] = jnp.concatenate(
        [x, jnp.zeros((x.shape[0], C_OUT), jnp.float32)], axis=-1)


def _stats_kernel(x_ref, s_ref, q_ref):
    x = x_ref[...]
    s_ref[0, ...] = jnp.sum(x, axis=0, keepdims=True)
    q_ref[0, ...] = jnp.sum(x * x, axis=0, keepdims=True)


def _mm_kernel(g_ref, w_ref, o_ref):
    w = w_ref[0]
    g = g_ref[0][:, :w.shape[0]]
    o_ref[0] = jnp.dot(g, w, preferred_element_type=jnp.float32)


def _mm_fold_kernel(g_ref, w_ref, s_ref, q_ref, gw_ref, bw_ref, o_ref):
    mu = jnp.sum(s_ref[...], axis=0) / N
    q = jnp.sum(q_ref[...], axis=0) / N
    iv = lax.rsqrt(q - mu * mu + EPS)
    scale = iv * gw_ref[...]
    shift = bw_ref[...] - mu * scale
    g = g_ref[0][:, :C_OUT]
    g = jnp.where(g >= 0, g, 0.01 * g)
    y = g * scale + shift
    o_ref[0] = jnp.dot(y, w_ref[0], preferred_element_type=jnp.float32)


def _apply2_kernel(x1_ref, x2_ref, s1_ref, q1_ref, s2_ref, q2_ref,
                   g1_ref, b1_ref, g2_ref, b2_ref, o_ref):
    def stats(s_ref, q_ref):
        mu = jnp.sum(s_ref[...], axis=0) / N
        q = jnp.sum(q_ref[...], axis=0) / N
        return mu, lax.rsqrt(q - mu * mu + EPS)

    mu1, iv1 = stats(s1_ref, q1_ref)
    mu2, iv2 = stats(s2_ref, q2_ref)
    y = ((x1_ref[...] - mu1) * iv1 * g1_ref[...] + b1_ref[...]
         + (x2_ref[...] - mu2) * iv2 * g2_ref[...] + b2_ref[...])
    o_ref[...] = jnp.concatenate(
        [y, jnp.zeros((y.shape[0], C_OUT), jnp.float32)], axis=-1)


def _matmul(g, w, K, fold=None):
    cin_g = g.shape[-1]
    cin_w = w.shape[1]
    in_specs = [
        pl.BlockSpec((1, PP, cin_g), lambda k: (k, 0, 0)),
        pl.BlockSpec((1, cin_w, C_OUT), lambda k: (k, 0, 0)),
    ]
    if fold is None:
        body = _mm_kernel
        args = (g, w)
    else:
        s, q, gw, bw = fold
        body = _mm_fold_kernel
        in_specs += [
            pl.BlockSpec((NB, 1, C_OUT), lambda k: (0, 0, 0)),
            pl.BlockSpec((NB, 1, C_OUT), lambda k: (0, 0, 0)),
            pl.BlockSpec((1, C_OUT), lambda k: (0, 0)),
            pl.BlockSpec((1, C_OUT), lambda k: (0, 0)),
        ]
        args = (g, w, s, q, gw.reshape(1, -1), bw.reshape(1, -1))
    out = pl.pallas_call(
        body,
        grid=(K,),
        in_specs=in_specs,
        out_specs=pl.BlockSpec((1, PP, C_OUT), lambda k: (k, 0, 0)),
        out_shape=jax.ShapeDtypeStruct((K, PP, C_OUT), jnp.float32),
    )(*args)
    return out.reshape(K * PP, C_OUT)


def _stats(x, repack=False):
    outs = [pl.BlockSpec((1, 1, C_OUT), lambda i: (i, 0, 0))] * 2
    shapes = [jax.ShapeDtypeStruct((NB, 1, C_OUT), jnp.float32)] * 2
    body = _stats_kernel
    if repack:
        outs = outs + [pl.BlockSpec((SBLK, C_IN), lambda i: (i, 0))]
        shapes = shapes + [jax.ShapeDtypeStruct((N, C_IN), jnp.float32)]
        body = _stats_repack_kernel
    return pl.pallas_call(
        body,
        grid=(NB,),
        in_specs=[pl.BlockSpec((SBLK, C_OUT), lambda i: (i, 0))],
        out_specs=outs,
        out_shape=shapes,
    )(x)


def _pad_flat(a, pad_value):
    a = a.astype(jnp.int32)
    return jnp.pad(a, ((0, 0), (0, PP - P)), constant_values=pad_value).reshape(-1)


def kernel(x_features, skip_features, W_trans, bn_t_g, bn_t_b, W1, bn1_g, bn1_b, W2, bn2_g, bn2_b, W_up, rb_trans_in, rb_trans_out, rb1_in, rb1_out, rb2_in, rb2_out, rb_up_in, rb_up_out):
    ti = _pad_flat(rb_trans_in, 0)
    to = _pad_flat(rb_trans_out, N)
    i1 = _pad_flat(rb1_in, 0)
    o1 = _pad_flat(rb1_out, N)
    i2 = _pad_flat(rb2_in, 0)
    o2 = _pad_flat(rb2_out, N)
    i12 = jnp.concatenate([i1, i2])
    ui = _pad_flat(rb_up_in, 0)
    uo = _pad_flat(rb_up_out, N_OUT)

    x = pl.pallas_call(
        _add_kernel,
        grid=(NB,),
        in_specs=[pl.BlockSpec((SBLK, C_IN), lambda i: (i, 0))] * 2,
        out_specs=pl.BlockSpec((SBLK, C_IN), lambda i: (i, 0)),
        out_shape=jax.ShapeDtypeStruct((N, C_IN), jnp.float32),
    )(x_features, skip_features)

    g_t = _gather(x, ti).reshape(27, PP, C_IN)
    m_t = _matmul(g_t, W_trans, 27)
    upa = jnp.zeros((N, C_OUT), jnp.float32).at[to].add(m_t)

    s_t, q_t, upa128 = _stats(upa, repack=True)

    w12 = jnp.concatenate([W1, W2], axis=0)
    g12 = _gather(upa128, i12).reshape(18, PP, C_IN)
    m12 = _matmul(g12, w12, 18, fold=(s_t, q_t, bn_t_g, bn_t_b))
    s1 = jnp.zeros((N, C_OUT), jnp.float32).at[o1].add(m12[:9 * PP])
    s2 = jnp.zeros((N, C_OUT), jnp.float32).at[o2].add(m12[9 * PP:])

    s_1, q_1 = _stats(s1)
    s_2, q_2 = _stats(s2)
    upe128 = pl.pallas_call(
        _apply2_kernel,
        grid=(NB,),
        in_specs=[pl.BlockSpec((SBLK, C_OUT), lambda i: (i, 0))] * 2
        + [pl.BlockSpec((NB, 1, C_OUT), lambda i: (0, 0, 0))] * 4
        + [pl.BlockSpec((1, C_OUT), lambda i: (0, 0))] * 4,
        out_specs=pl.BlockSpec((SBLK, C_IN), lambda i: (i, 0)),
        out_shape=jax.ShapeDtypeStruct((N, C_IN), jnp.float32),
    )(s1, s2, s_1, q_1, s_2, q_2, bn1_g.reshape(1, -1), bn1_b.reshape(1, -1),
      bn2_g.reshape(1, -1), bn2_b.reshape(1, -1))

    g_u = _gather(upe128, ui).reshape(27, PP, C_IN)
    m_u = _matmul(g_u, W_up, 27)
    return jnp.zeros((N_OUT, C_OUT), jnp.float32).at[uo].add(m_u)
